# Initial kernel scaffold; baseline (speedup 1.0000x reference)
import functools

import jax
import jax.numpy as jnp
from jax import lax
from jax.experimental import pallas as pl
from jax.experimental.pallas import tpu as pltpu
from jax.experimental.pallas import tpu_sc as plsc

N = 10000
E = 320000
R = 8
D = 128
NEG = 0.2

NC = 2
NS = 16
NW = NC * NS
EW = E // NW
CH = 400
NCHUNK = EW // CH
SUB = 80
NSUB = CH // SUB
NV = SUB // 16
ROWS_T = N // NS
ZR = 125

_HI = lax.Precision.HIGHEST


BN = 1000
NB = N // BN


def _txw_body(x_ref, w_ref, o_ref):
    o_ref[...] = jnp.dot(x_ref[...], w_ref[0],
                         preferred_element_type=jnp.float32, precision=_HI)


def _txw(x, W):
    return pl.pallas_call(
        _txw_body,
        grid=(R, NB),
        in_specs=[
            pl.BlockSpec((BN, D), lambda r, i: (i, 0)),
            pl.BlockSpec((1, D, D), lambda r, i: (r, 0, 0)),
        ],
        out_specs=pl.BlockSpec((BN, D), lambda r, i: (r * NB + i, 0)),
        out_shape=jax.ShapeDtypeStruct((R * N, D), jnp.float32),
    )(x, W)


def _tqk_body(x_ref, w_ref, q_ref, k_ref, qk_ref, sh_ref, scr):
    i = pl.program_id(0)
    cols = [jnp.dot(w_ref[r], q_ref[...],
                    preferred_element_type=jnp.float32, precision=_HI)
            for r in range(R)]
    cols += [jnp.dot(w_ref[r], k_ref[...],
                     preferred_element_type=jnp.float32, precision=_HI)
             for r in range(R)]
    QK = jnp.concatenate(cols, axis=1)
    blk = jnp.dot(x_ref[...], QK,
                  preferred_element_type=jnp.float32, precision=_HI)
    qk_ref[...] = blk
    mq = jnp.max(blk[:, :R])
    mk = jnp.max(blk[:, R:])

    @pl.when(i == 0)
    def _():
        scr[0] = mq
        scr[1] = mk

    @pl.when(i > 0)
    def _():
        scr[0] = jnp.maximum(scr[0], mq)
        scr[1] = jnp.maximum(scr[1], mk)

    @pl.when(i == pl.num_programs(0) - 1)
    def _():
        m = scr[0] + scr[1]
        sh_ref[0, 0] = jnp.where(m > 0, m, m * NEG)


def _tqk(x, W, q, k):
    return pl.pallas_call(
        _tqk_body,
        grid=(NB,),
        in_specs=[
            pl.BlockSpec((BN, D), lambda i: (i, 0)),
            pl.BlockSpec((R, D, D), lambda i: (0, 0, 0)),
            pl.BlockSpec((D, 1), lambda i: (0, 0)),
            pl.BlockSpec((D, 1), lambda i: (0, 0)),
        ],
        out_specs=[
            pl.BlockSpec((BN, 16), lambda i: (i, 0)),
            pl.BlockSpec((1, 1), lambda i: (0, 0)),
        ],
        out_shape=[
            jax.ShapeDtypeStruct((N, 16), jnp.float32),
            jax.ShapeDtypeStruct((1, 1), jnp.float32),
        ],
        scratch_shapes=[pltpu.SMEM((2,), jnp.float32)],
    )(x, W, q, k)


def _mid_body(a_ref, d_ref, b_ref, o_ref):
    a = a_ref[0] + a_ref[1]
    den = d_ref[0, :, 0:1] + d_ref[1, :, 0:1]
    o_ref[...] = jnp.maximum(a / (den + 1e-16) + b_ref[...], 0.0)


def _middle(acc, den, b):
    return pl.pallas_call(
        _mid_body,
        grid=(NB,),
        in_specs=[
            pl.BlockSpec((2, BN, D), lambda i: (0, i, 0)),
            pl.BlockSpec((2, BN, 16), lambda i: (0, i, 0)),
            pl.BlockSpec((1, D), lambda i: (0, 0)),
        ],
        out_specs=pl.BlockSpec((BN, D), lambda i: (i, 0)),
        out_shape=jax.ShapeDtypeStruct((N, D), jnp.float32),
    )(acc, den, b)


def _fin_body(a_ref, d_ref, b_ref, w1_ref, b1_ref, w2_ref, b2_ref, o_ref, scr):
    i = pl.program_id(0)
    a = a_ref[0] + a_ref[1]
    den = d_ref[0, :, 0:1] + d_ref[1, :, 0:1]
    h = jnp.maximum(a / (den + 1e-16) + b_ref[...], 0.0)
    s = jnp.sum(h, axis=0, keepdims=True)
    m = jnp.max(h, axis=0, keepdims=True)

    @pl.when(i == 0)
    def _():
        scr[0:1] = s
        scr[1:2] = m

    @pl.when(i > 0)
    def _():
        scr[0:1] = scr[0:1] + s
        scr[1:2] = jnp.maximum(scr[1:2], m)

    @pl.when(i == pl.num_programs(0) - 1)
    def _():
        avg = scr[0:1] / float(N)
        mx = scr[1:2]
        g = jnp.tanh(jnp.concatenate([avg, mx], axis=1))
        z = lax.dot_general(g, w1_ref[...], (((1,), (1,)), ((), ())),
                            preferred_element_type=jnp.float32, precision=_HI)
        z = jnp.maximum(z + b1_ref[...], 0.0)
        o = lax.dot_general(z, w2_ref[...], (((1,), (1,)), ((), ())),
                            preferred_element_type=jnp.float32, precision=_HI)
        o_ref[...] = jax.nn.sigmoid(o + b2_ref[...])


def _final(acc, den, b, fc1_w, fc1_b, fc2_w, fc2_b):
    return pl.pallas_call(
        _fin_body,
        grid=(NB,),
        in_specs=[
            pl.BlockSpec((2, BN, D), lambda i: (0, i, 0)),
            pl.BlockSpec((2, BN, 16), lambda i: (0, i, 0)),
            pl.BlockSpec((1, D), lambda i: (0, 0)),
            pl.BlockSpec((D, 2 * D), lambda i: (0, 0)),
            pl.BlockSpec((1, D), lambda i: (0, 0)),
            pl.BlockSpec((1, D), lambda i: (0, 0)),
            pl.BlockSpec((1, 1), lambda i: (0, 0)),
        ],
        out_specs=pl.BlockSpec((1, 1), lambda i: (0, 0)),
        out_shape=jax.ShapeDtypeStruct((1, 1), jnp.float32),
        scratch_shapes=[pltpu.VMEM((2, D), jnp.float32)],
    )(acc, den, b, fc1_w, fc1_b, fc2_w, fc2_b)


def _sc_body(src_h, dst_h, et_h, xw_h, xqk_h, shift_h,
             acc_o, den_o,
             srcv, dstv, etv, qidx, kidx, ridx, dsti,
             qiv, kjv, exr, rows, zbuf, zden, shiftv,
             acc_sh, den_sh):
    cid = lax.axis_index("c")
    sid = lax.axis_index("s")
    wid = sid * NC + cid
    base = wid * EW
    r0 = sid * ROWS_T

    zero16 = jnp.zeros((16,), jnp.float32)

    def zfill(i, c0):
        for c in range(D // 16):
            zbuf[i, pl.ds(c * 16, 16)] = zero16
        zden[i, :] = zero16
        return c0

    lax.fori_loop(0, ZR, zfill, 0)

    def zcopy(c, c0):
        off = r0 + c * ZR
        pltpu.sync_copy(zbuf, acc_sh.at[pl.ds(off, ZR)])
        pltpu.sync_copy(zden, den_sh.at[pl.ds(off, ZR)])
        return c0

    lax.fori_loop(0, ROWS_T // ZR, zcopy, 0)
    plsc.subcore_barrier()

    pltpu.sync_copy(shift_h, shiftv)
    sh = shiftv[...]

    def chunk(t, c0):
        cb = pl.multiple_of(base + t * CH, CH)
        pltpu.sync_copy(src_h.at[pl.ds(cb, CH)], srcv)
        pltpu.sync_copy(dst_h.at[pl.ds(cb, CH)], dstv)
        pltpu.sync_copy(et_h.at[pl.ds(cb, CH)], etv)

        def subj(j, c1):
            def vecs(l, c2):
                off = pl.multiple_of(j * SUB + l * 16, 16)
                s16 = srcv[pl.ds(off, 16)]
                d16 = dstv[pl.ds(off, 16)]
                t16 = etv[pl.ds(off, 16)]
                lo = pl.multiple_of(l * 16, 16)
                qidx[j, pl.ds(lo, 16)] = d16 * 16 + t16
                kidx[j, pl.ds(lo, 16)] = s16 * 16 + (t16 + 8)
                ridx[j, pl.ds(lo, 16)] = t16 * N + s16
                dsti[j, pl.ds(lo, 16)] = d16
                return c2

            lax.fori_loop(0, NV, vecs, 0)

            pltpu.sync_copy(xqk_h.at[qidx.at[j]], qiv.at[j])
            pltpu.sync_copy(xqk_h.at[kidx.at[j]], kjv.at[j])
            pltpu.sync_copy(xw_h.at[ridx.at[j]], rows)

            def vl(l, c2):
                lo = pl.multiple_of(l * 16, 16)
                q16 = qiv[j, pl.ds(lo, 16)]
                k16 = kjv[j, pl.ds(lo, 16)]
                aval = q16 + k16
                aval = jnp.where(aval > 0, aval, aval * NEG)
                e16 = jnp.exp(aval - sh)
                for lane in range(16):
                    row = l * 16 + lane
                    sp = jnp.broadcast_to(
                        lax.slice(e16, (lane,), (lane + 1,)), (16,))
                    exr[row, :] = sp
                    for c in range(D // 16):
                        rows[row, pl.ds(c * 16, 16)] = (
                            rows[row, pl.ds(c * 16, 16)] * sp)
                return c2

            lax.fori_loop(0, NV, vl, 0)

            pltpu.sync_copy(rows, acc_sh.at[dsti.at[j]], add=True)
            pltpu.sync_copy(exr, den_sh.at[dsti.at[j]], add=True)
            return c1

        lax.fori_loop(0, NSUB, subj, 0)
        return c0

    lax.fori_loop(0, NCHUNK, chunk, 0)
    plsc.subcore_barrier()

    pltpu.sync_copy(acc_sh.at[pl.ds(r0, ROWS_T)],
                    acc_o.at[cid, pl.ds(r0, ROWS_T)])
    pltpu.sync_copy(den_sh.at[pl.ds(r0, ROWS_T)],
                    den_o.at[cid, pl.ds(r0, ROWS_T)])


_sc_layer = pl.kernel(
    _sc_body,
    out_type=[
        jax.ShapeDtypeStruct((NC, N, D), jnp.float32),
        jax.ShapeDtypeStruct((NC, N, 16), jnp.float32),
    ],
    mesh=plsc.VectorSubcoreMesh(core_axis_name="c", subcore_axis_name="s"),
    scratch_types=[
        pltpu.VMEM((CH,), jnp.int32),
        pltpu.VMEM((CH,), jnp.int32),
        pltpu.VMEM((CH,), jnp.int32),
        pltpu.VMEM((NSUB, SUB), jnp.int32),
        pltpu.VMEM((NSUB, SUB), jnp.int32),
        pltpu.VMEM((NSUB, SUB), jnp.int32),
        pltpu.VMEM((NSUB, SUB), jnp.int32),
        pltpu.VMEM((NSUB, SUB), jnp.float32),
        pltpu.VMEM((NSUB, SUB), jnp.float32),
        pltpu.VMEM((SUB, 16), jnp.float32),
        pltpu.VMEM((SUB, D), jnp.float32),
        pltpu.VMEM((ZR, D), jnp.float32),
        pltpu.VMEM((ZR, 16), jnp.float32),
        pltpu.VMEM((16,), jnp.float32),
        pltpu.VMEM_SHARED((N, D), jnp.float32),
        pltpu.VMEM_SHARED((N, 16), jnp.float32),
    ],
)


def kernel(x, edge_index, edge_type, W1, q1, k1, b1,
           W2, q2, k2, b2, fc1_w, fc1_b, fc2_w, fc2_b):
    src = edge_index[0]
    dst = edge_index[1]
    et = edge_type

    xw1 = _txw(x, W1)
    xqk1, sh1 = _tqk(x, W1, q1, k1)
    sh1v = jnp.broadcast_to(sh1.reshape(1), (16,))
    acc1, den1 = _sc_layer(src, dst, et, xw1, xqk1.reshape(-1), sh1v)
    h = _middle(acc1, den1, b1.reshape(1, D))

    xw2 = _txw(h, W2)
    xqk2, sh2 = _tqk(h, W2, q2, k2)
    sh2v = jnp.broadcast_to(sh2.reshape(1), (16,))
    acc2, den2 = _sc_layer(src, dst, et, xw2, xqk2.reshape(-1), sh2v)

    out = _final(acc2, den2, b2.reshape(1, D), fc1_w, fc1_b.reshape(1, D),
                 fc2_w, fc2_b.reshape(1, 1))
    return out.reshape(1)


# speedup vs baseline: 24.7341x; 24.7341x over previous
import jax
import jax.numpy as jnp
from jax import lax
from jax.experimental import pallas as pl
from jax.experimental.pallas import tpu as pltpu
from jax.experimental.pallas import tpu_sc as plsc

N = 10000
N2 = 10240
E = 320000
R = 8
D = 128
NEG = 0.2

NC = 2
NS = 16
NW = NC * NS
EW = E // NW
CH = 400
NCHUNK = EW // CH
SUB = 80
NSUB = CH // SUB
NV = SUB // 16
ROWS_T = N2 // NS
ZR = 128

_HI = lax.Precision.HIGHEST


BN = 1280
NB = N2 // BN
BM = 640
NM = N2 // BM


def _txw_body(x_ref, w_ref, o_ref):
    o_ref[...] = jnp.dot(x_ref[...], w_ref[0],
                         preferred_element_type=jnp.float32, precision=_HI)


def _txw(x, W):
    return pl.pallas_call(
        _txw_body,
        grid=(R, NB),
        in_specs=[
            pl.BlockSpec((BN, D), lambda r, i: (i, 0)),
            pl.BlockSpec((1, D, D), lambda r, i: (r, 0, 0)),
        ],
        out_specs=pl.BlockSpec((BN, D), lambda r, i: (r * NB + i, 0)),
        out_shape=jax.ShapeDtypeStruct((R * N2, D), jnp.float32),
    )(x, W)


def _tqk_body(x_ref, w_ref, q_ref, k_ref, qk_ref, sh_ref, scr):
    i = pl.program_id(0)
    cols = [jnp.dot(w_ref[r], q_ref[...],
                    preferred_element_type=jnp.float32, precision=_HI)
            for r in range(R)]
    cols += [jnp.dot(w_ref[r], k_ref[...],
                     preferred_element_type=jnp.float32, precision=_HI)
             for r in range(R)]
    QK = jnp.concatenate(cols, axis=1)
    blk = jnp.dot(x_ref[...], QK,
                  preferred_element_type=jnp.float32, precision=_HI)
    qk_ref[...] = blk
    mq = jnp.max(blk[:, :R])
    mk = jnp.max(blk[:, R:])

    @pl.when(i == 0)
    def _():
        scr[0] = mq
        scr[1] = mk

    @pl.when(i > 0)
    def _():
        scr[0] = jnp.maximum(scr[0], mq)
        scr[1] = jnp.maximum(scr[1], mk)

    @pl.when(i == pl.num_programs(0) - 1)
    def _():
        m = scr[0] + scr[1]
        sh_ref[...] = jnp.where(m > 0, m, m * NEG).reshape(1, 1)


def _tqk(x, W, q, k):
    return pl.pallas_call(
        _tqk_body,
        grid=(NB,),
        in_specs=[
            pl.BlockSpec((BN, D), lambda i: (i, 0)),
            pl.BlockSpec((R, D, D), lambda i: (0, 0, 0)),
            pl.BlockSpec((D, 1), lambda i: (0, 0)),
            pl.BlockSpec((D, 1), lambda i: (0, 0)),
        ],
        out_specs=[
            pl.BlockSpec((BN, 16), lambda i: (i, 0)),
            pl.BlockSpec((1, 1), lambda i: (0, 0)),
        ],
        out_shape=[
            jax.ShapeDtypeStruct((N2, 16), jnp.float32),
            jax.ShapeDtypeStruct((1, 1), jnp.float32),
        ],
        scratch_shapes=[pltpu.SMEM((2,), jnp.float32)],
    )(x, W, q, k)


def _mid_body(a_ref, d_ref, b_ref, o_ref):
    a = a_ref[0] + a_ref[1]
    den = d_ref[0] + d_ref[1]
    o_ref[...] = jnp.maximum(a / (den + 1e-16) + b_ref[...], 0.0)


def _middle(acc, den, b):
    return pl.pallas_call(
        _mid_body,
        grid=(NM,),
        in_specs=[
            pl.BlockSpec((2, BM, D), lambda i: (0, i, 0)),
            pl.BlockSpec((2, BM, 1), lambda i: (0, i, 0)),
            pl.BlockSpec((1, D), lambda i: (0, 0)),
        ],
        out_specs=pl.BlockSpec((BM, D), lambda i: (i, 0)),
        out_shape=jax.ShapeDtypeStruct((N2, D), jnp.float32),
    )(acc, den, b)


def _fin_body(a_ref, d_ref, b_ref, w1_ref, b1_ref, w2_ref, b2_ref, o_ref, scr):
    i = pl.program_id(0)
    a = a_ref[0] + a_ref[1]
    den = d_ref[0] + d_ref[1]
    h = jnp.maximum(a / (den + 1e-16) + b_ref[...], 0.0)
    rid = lax.broadcasted_iota(jnp.int32, (BM, 1), 0) + i * BM
    h = jnp.where(rid < N, h, 0.0)
    s = jnp.sum(h, axis=0, keepdims=True)
    m = jnp.max(h, axis=0, keepdims=True)

    @pl.when(i == 0)
    def _():
        scr[0:1] = s
        scr[1:2] = m

    @pl.when(i > 0)
    def _():
        scr[0:1] = scr[0:1] + s
        scr[1:2] = jnp.maximum(scr[1:2], m)

    @pl.when(i == pl.num_programs(0) - 1)
    def _():
        avg = scr[0:1] / float(N)
        mx = scr[1:2]
        g = jnp.tanh(jnp.concatenate([avg, mx], axis=1))
        z = jnp.sum(w1_ref[...] * g, axis=1, keepdims=True)
        z = jnp.maximum(z + b1_ref[...], 0.0)
        o = jnp.sum(z * w2_ref[...], axis=0, keepdims=True) + b2_ref[...]
        o_ref[...] = jax.nn.sigmoid(o)


def _final(acc, den, b, fc1_w, fc1_b, fc2_w, fc2_b):
    return pl.pallas_call(
        _fin_body,
        grid=(NM,),
        in_specs=[
            pl.BlockSpec((2, BM, D), lambda i: (0, i, 0)),
            pl.BlockSpec((2, BM, 1), lambda i: (0, i, 0)),
            pl.BlockSpec((1, D), lambda i: (0, 0)),
            pl.BlockSpec((D, 2 * D), lambda i: (0, 0)),
            pl.BlockSpec((D, 1), lambda i: (0, 0)),
            pl.BlockSpec((D, 1), lambda i: (0, 0)),
            pl.BlockSpec((1, 1), lambda i: (0, 0)),
        ],
        out_specs=pl.BlockSpec((1, 1), lambda i: (0, 0)),
        out_shape=jax.ShapeDtypeStruct((1, 1), jnp.float32),
        scratch_shapes=[pltpu.VMEM((2, D), jnp.float32)],
    )(acc, den, b, fc1_w, fc1_b, fc2_w, fc2_b)


def _sc_body(src_h, dst_h, et_h, xw_h, xqk_h, shift_h,
             acc_o, den_o,
             srcv, dstv, etv, qidx, kidx, ridx, dsti,
             qiv, kjv, exv, rows, zv, shiftv,
             acc_sh, den_sh):
    cid = lax.axis_index("c")
    sid = lax.axis_index("s")
    wid = sid * NC + cid
    base = wid * EW
    r0 = sid * ROWS_T

    zero16 = jnp.zeros((16,), jnp.float32)

    def zfill(i, c0):
        for c in range(D // 16):
            rows[i, pl.ds(c * 16, 16)] = zero16
        return c0

    lax.fori_loop(0, SUB, zfill, 0)

    def zfill2(i, c0):
        zv[pl.ds(i * 16, 16)] = zero16
        return c0

    lax.fori_loop(0, ROWS_T // 16, zfill2, 0)

    def zcopy(c, c0):
        off = pl.multiple_of(r0 + c * SUB, 16)
        pltpu.sync_copy(rows, acc_sh.at[pl.ds(off, SUB)])
        return c0

    lax.fori_loop(0, ROWS_T // SUB, zcopy, 0)
    pltpu.sync_copy(zv, den_sh.at[pl.ds(r0, ROWS_T)])
    plsc.subcore_barrier()

    pltpu.sync_copy(shift_h, shiftv)
    sh = shiftv[...]

    def chunk(t, c0):
        cb = pl.multiple_of(base + t * CH, CH)
        pltpu.sync_copy(src_h.at[pl.ds(cb, CH)], srcv)
        pltpu.sync_copy(dst_h.at[pl.ds(cb, CH)], dstv)
        pltpu.sync_copy(et_h.at[pl.ds(cb, CH)], etv)

        def subj(j, c1):
            def vecs(l, c2):
                off = pl.multiple_of(j * SUB + l * 16, 16)
                s16 = srcv[pl.ds(off, 16)]
                d16 = dstv[pl.ds(off, 16)]
                t16 = etv[pl.ds(off, 16)]
                lo = pl.multiple_of(l * 16, 16)
                qidx[j, pl.ds(lo, 16)] = d16 * 16 + t16
                kidx[j, pl.ds(lo, 16)] = s16 * 16 + (t16 + 8)
                ridx[j, pl.ds(lo, 16)] = t16 * N2 + s16
                dsti[j, pl.ds(lo, 16)] = d16
                return c2

            lax.fori_loop(0, NV, vecs, 0)

            pltpu.sync_copy(xqk_h.at[qidx.at[j]], qiv.at[j])
            pltpu.sync_copy(xqk_h.at[kidx.at[j]], kjv.at[j])
            pltpu.sync_copy(xw_h.at[ridx.at[j]], rows)

            def vl(l, c2):
                lo = pl.multiple_of(l * 16, 16)
                q16 = qiv[j, pl.ds(lo, 16)]
                k16 = kjv[j, pl.ds(lo, 16)]
                aval = q16 + k16
                aval = jnp.where(aval > 0, aval, aval * NEG)
                e16 = jnp.exp(aval - sh)
                exv[j, pl.ds(lo, 16)] = e16
                for lane in range(16):
                    row = l * 16 + lane
                    sp = jnp.broadcast_to(
                        lax.slice(e16, (lane,), (lane + 1,)), (16,))
                    for c in range(D // 16):
                        rows[row, pl.ds(c * 16, 16)] = (
                            rows[row, pl.ds(c * 16, 16)] * sp)
                return c2

            lax.fori_loop(0, NV, vl, 0)

            pltpu.sync_copy(rows, acc_sh.at[dsti.at[j]], add=True)
            pltpu.sync_copy(exv.at[j], den_sh.at[dsti.at[j]], add=True)
            return c1

        lax.fori_loop(0, NSUB, subj, 0)
        return c0

    lax.fori_loop(0, NCHUNK, chunk, 0)
    plsc.subcore_barrier()

    pltpu.sync_copy(acc_sh.at[pl.ds(r0, ROWS_T)],
                    acc_o.at[cid, pl.ds(r0, ROWS_T)])
    off2 = pl.multiple_of(cid * N2 + r0, 128)
    pltpu.sync_copy(den_sh.at[pl.ds(r0, ROWS_T)],
                    den_o.at[pl.ds(off2, ROWS_T)])


_sc_layer = pl.kernel(
    _sc_body,
    out_type=[
        jax.ShapeDtypeStruct((NC, N2, D), jnp.float32),
        jax.ShapeDtypeStruct((NC * N2,), jnp.float32),
    ],
    mesh=plsc.VectorSubcoreMesh(core_axis_name="c", subcore_axis_name="s"),
    scratch_types=[
        pltpu.VMEM((CH,), jnp.int32),
        pltpu.VMEM((CH,), jnp.int32),
        pltpu.VMEM((CH,), jnp.int32),
        pltpu.VMEM((NSUB, SUB), jnp.int32),
        pltpu.VMEM((NSUB, SUB), jnp.int32),
        pltpu.VMEM((NSUB, SUB), jnp.int32),
        pltpu.VMEM((NSUB, SUB), jnp.int32),
        pltpu.VMEM((NSUB, SUB), jnp.float32),
        pltpu.VMEM((NSUB, SUB), jnp.float32),
        pltpu.VMEM((NSUB, SUB), jnp.float32),
        pltpu.VMEM((SUB, D), jnp.float32),
        pltpu.VMEM((ROWS_T,), jnp.float32),
        pltpu.VMEM((16,), jnp.float32),
        pltpu.VMEM_SHARED((N2, D), jnp.float32),
        pltpu.VMEM_SHARED((N2,), jnp.float32),
    ],
)


def kernel(x, edge_index, edge_type, W1, q1, k1, b1,
           W2, q2, k2, b2, fc1_w, fc1_b, fc2_w, fc2_b):
    src = edge_index[0]
    dst = edge_index[1]
    et = edge_type

    x2 = jnp.pad(x, ((0, N2 - N), (0, 0)))

    xw1 = _txw(x2, W1)
    xqk1, sh1 = _tqk(x2, W1, q1, k1)
    sh1v = jnp.broadcast_to(sh1.reshape(1), (16,))
    acc1, den1 = _sc_layer(src, dst, et, xw1, xqk1.reshape(-1), sh1v)
    h = _middle(acc1, den1.reshape(NC, N2, 1), b1.reshape(1, D))

    xw2 = _txw(h, W2)
    xqk2, sh2 = _tqk(h, W2, q2, k2)
    sh2v = jnp.broadcast_to(sh2.reshape(1), (16,))
    acc2, den2 = _sc_layer(src, dst, et, xw2, xqk2.reshape(-1), sh2v)

    out = _final(acc2, den2.reshape(NC, N2, 1), b2.reshape(1, D), fc1_w, fc1_b.reshape(D, 1),
                 fc2_w.reshape(D, 1), fc2_b.reshape(1, 1))
    return out.reshape(1)
